# baseline (speedup 1.0000x reference)
import jax
import jax.numpy as jnp
from jax.experimental import pallas as pl
from jax.experimental.pallas import tpu as pltpu

POOL = 32
PLEN = 32
NCTX = 32
ED = 768
TOPK = 5
NCLS = 100
SUF = 12
NTOK = 1 + PLEN + NCTX + SUF
G = 10
NG = NCLS // G
NBUF = 3


def _body(x_ref, key_ref, pen_ref, flag_ref, w_ref, b_ref, ctx_ref,
          pre_ref, suf_ref, pool_hbm,
          out_hbm, pool_out, key_out,
          ring_ref, pool_ref, fetch_sem, copy_sem, out_sem):
    pool_fetch = pltpu.make_async_copy(pool_hbm, pool_ref, fetch_sem)
    pool_fetch.start()
    cp_key = pltpu.make_async_copy(key_ref, key_out, copy_sem)
    cp_key.start()

    x = jnp.mean(x_ref[...], axis=0)
    key = key_ref[...]
    dots = jnp.sum(key * x[None, :], axis=1)
    inv = jax.lax.rsqrt(jnp.sum(key * key, axis=1))
    s = dots * inv
    s = jnp.where(flag_ref[0, 0] != 0, s * pen_ref[0, :], s)
    si = s[:, None]
    sj = s[None, :]
    ii = jax.lax.broadcasted_iota(jnp.int32, (POOL, POOL), 0)
    jj = jax.lax.broadcasted_iota(jnp.int32, (POOL, POOL), 1)
    beats = (sj > si) | ((sj == si) & (jj < ii))
    rank = jnp.sum(beats.astype(jnp.int32), axis=1)
    iota = jax.lax.broadcasted_iota(jnp.int32, (1, POOL), 1)[0]

    pool_fetch.wait()
    cp_pool = pltpu.make_async_copy(pool_ref, pool_out, copy_sem)
    cp_pool.start()

    w = w_ref[0, :]
    b = b_ref[0, 0]
    comb = jnp.zeros((PLEN, ED), jnp.float32)
    for k in range(TOPK):
        idx_k = jnp.sum(jnp.where(rank == k, iota, 0))
        sel = pool_ref[pl.ds(idx_k, 1), :, :][0]
        z = jnp.sum(sel * w[None, :], axis=-1) + b
        alpha = 1.0 / (1.0 + jnp.exp(-z))
        comb = comb + alpha[:, None] * sel

    mid = jnp.concatenate([comb, ctx_ref[...]], axis=0)
    midb = jnp.broadcast_to(mid[None], (G, PLEN + NCTX, ED))
    for buf in range(NBUF):
        ring_ref[buf, :, 1:1 + PLEN + NCTX, :] = midb

    dmas = [None] * NG
    for grp in range(NG):
        slot = grp % NBUF
        if grp >= NBUF:
            dmas[grp - NBUF].wait()
        ring_ref[slot, :, 0:1, :] = pre_ref[pl.ds(grp * G, G)]
        ring_ref[slot, :, 1 + PLEN + NCTX:, :] = suf_ref[pl.ds(grp * G, G)]
        d = pltpu.make_async_copy(ring_ref.at[slot],
                                  out_hbm.at[pl.ds(grp * G, G)],
                                  out_sem.at[slot])
        d.start()
        dmas[grp] = d
    for grp in range(NG - NBUF, NG):
        dmas[grp].wait()
    cp_pool.wait()
    cp_key.wait()


@jax.jit
def _run(x_embed, prompt_pool, prompt_key, ctx, w_alpha, b_alpha,
         token_prefix, token_suffix, penalty_factors, train_flag):
    pen2 = penalty_factors.reshape(1, POOL)
    flag2 = jnp.asarray(train_flag, jnp.int32).reshape(1, 1)
    b2 = b_alpha.reshape(1, 1)
    vmem = pl.BlockSpec(memory_space=pltpu.MemorySpace.VMEM)
    hbm = pl.BlockSpec(memory_space=pltpu.MemorySpace.HBM)
    return pl.pallas_call(
        _body,
        in_specs=[vmem, vmem, vmem, vmem, vmem, vmem, vmem, vmem, vmem, hbm],
        out_specs=[hbm, hbm, hbm],
        out_shape=[
            jax.ShapeDtypeStruct((NCLS, NTOK, ED), jnp.float32),
            jax.ShapeDtypeStruct((POOL, PLEN, ED), jnp.float32),
            jax.ShapeDtypeStruct((POOL, ED), jnp.float32),
        ],
        scratch_shapes=[
            pltpu.VMEM((NBUF, G, NTOK, ED), jnp.float32),
            pltpu.VMEM((POOL, PLEN, ED), jnp.float32),
            pltpu.SemaphoreType.DMA,
            pltpu.SemaphoreType.DMA,
            pltpu.SemaphoreType.DMA((NBUF,)),
        ],
    )(x_embed, prompt_key, pen2, flag2, w_alpha, b2, ctx,
      token_prefix, token_suffix, prompt_pool)


def kernel(x_embed, prompt_pool, prompt_key, ctx, w_alpha, b_alpha,
           token_prefix, token_suffix, penalty_factors, train_flag):
    prompts, pool_out, key_out = _run(
        x_embed, prompt_pool, prompt_key, ctx, w_alpha, b_alpha,
        token_prefix, token_suffix, penalty_factors, train_flag)
    return (prompts, pool_out, key_out)


# speedup vs baseline: 1.0041x; 1.0041x over previous
import jax
import jax.numpy as jnp
from jax.experimental import pallas as pl
from jax.experimental.pallas import tpu as pltpu

POOL = 32
PLEN = 32
NCTX = 32
ED = 768
TOPK = 5
NCLS = 100
SUF = 12
NTOK = 1 + PLEN + NCTX + SUF
G = 4
NG = NCLS // G
NBUF = 4


def _body(x_ref, key_ref, pen_ref, flag_ref, w_ref, b_ref, ctx_ref,
          pre_ref, suf_ref, pool_hbm,
          out_hbm, pool_out, key_out,
          ring_ref, pool_ref, fetch_sem, copy_sem, out_sem):
    pool_fetch = pltpu.make_async_copy(pool_hbm, pool_ref, fetch_sem)
    pool_fetch.start()
    cp_key = pltpu.make_async_copy(key_ref, key_out, copy_sem)
    cp_key.start()

    x = jnp.mean(x_ref[...], axis=0)
    key = key_ref[...]
    dots = jnp.sum(key * x[None, :], axis=1)
    inv = jax.lax.rsqrt(jnp.sum(key * key, axis=1))
    s = dots * inv
    s = jnp.where(flag_ref[0, 0] != 0, s * pen_ref[0, :], s)
    si = s[:, None]
    sj = s[None, :]
    ii = jax.lax.broadcasted_iota(jnp.int32, (POOL, POOL), 0)
    jj = jax.lax.broadcasted_iota(jnp.int32, (POOL, POOL), 1)
    beats = (sj > si) | ((sj == si) & (jj < ii))
    rank = jnp.sum(beats.astype(jnp.int32), axis=1)
    iota = jax.lax.broadcasted_iota(jnp.int32, (1, POOL), 1)[0]

    pool_fetch.wait()
    cp_pool = pltpu.make_async_copy(pool_ref, pool_out, copy_sem)
    cp_pool.start()

    w = w_ref[0, :]
    b = b_ref[0, 0]
    comb = jnp.zeros((PLEN, ED), jnp.float32)
    for k in range(TOPK):
        idx_k = jnp.sum(jnp.where(rank == k, iota, 0))
        sel = pool_ref[pl.ds(idx_k, 1), :, :][0]
        z = jnp.sum(sel * w[None, :], axis=-1) + b
        alpha = 1.0 / (1.0 + jnp.exp(-z))
        comb = comb + alpha[:, None] * sel

    mid = jnp.concatenate([comb, ctx_ref[...]], axis=0)
    midb = jnp.broadcast_to(mid[None], (G, PLEN + NCTX, ED))
    for buf in range(NBUF):
        ring_ref[buf, :, 1:1 + PLEN + NCTX, :] = midb

    dmas = [None] * NG
    for grp in range(NG):
        slot = grp % NBUF
        if grp >= NBUF:
            dmas[grp - NBUF].wait()
        ring_ref[slot, :, 0:1, :] = pre_ref[pl.ds(grp * G, G)]
        ring_ref[slot, :, 1 + PLEN + NCTX:, :] = suf_ref[pl.ds(grp * G, G)]
        d = pltpu.make_async_copy(ring_ref.at[slot],
                                  out_hbm.at[pl.ds(grp * G, G)],
                                  out_sem.at[slot])
        d.start()
        dmas[grp] = d
    for grp in range(NG - NBUF, NG):
        dmas[grp].wait()
    cp_pool.wait()
    cp_key.wait()


@jax.jit
def _run(x_embed, prompt_pool, prompt_key, ctx, w_alpha, b_alpha,
         token_prefix, token_suffix, penalty_factors, train_flag):
    pen2 = penalty_factors.reshape(1, POOL)
    flag2 = jnp.asarray(train_flag, jnp.int32).reshape(1, 1)
    b2 = b_alpha.reshape(1, 1)
    vmem = pl.BlockSpec(memory_space=pltpu.MemorySpace.VMEM)
    hbm = pl.BlockSpec(memory_space=pltpu.MemorySpace.HBM)
    return pl.pallas_call(
        _body,
        in_specs=[vmem, vmem, vmem, vmem, vmem, vmem, vmem, vmem, vmem, hbm],
        out_specs=[hbm, hbm, hbm],
        out_shape=[
            jax.ShapeDtypeStruct((NCLS, NTOK, ED), jnp.float32),
            jax.ShapeDtypeStruct((POOL, PLEN, ED), jnp.float32),
            jax.ShapeDtypeStruct((POOL, ED), jnp.float32),
        ],
        scratch_shapes=[
            pltpu.VMEM((NBUF, G, NTOK, ED), jnp.float32),
            pltpu.VMEM((POOL, PLEN, ED), jnp.float32),
            pltpu.SemaphoreType.DMA,
            pltpu.SemaphoreType.DMA,
            pltpu.SemaphoreType.DMA((NBUF,)),
        ],
    )(x_embed, prompt_key, pen2, flag2, w_alpha, b2, ctx,
      token_prefix, token_suffix, prompt_pool)


def kernel(x_embed, prompt_pool, prompt_key, ctx, w_alpha, b_alpha,
           token_prefix, token_suffix, penalty_factors, train_flag):
    prompts, pool_out, key_out = _run(
        x_embed, prompt_pool, prompt_key, ctx, w_alpha, b_alpha,
        token_prefix, token_suffix, penalty_factors, train_flag)
    return (prompts, pool_out, key_out)
